# Initial kernel scaffold; baseline (speedup 1.0000x reference)
import jax
import jax.numpy as jnp
from jax.experimental import pallas as pl


def kernel(x, Wr, br, We, be):
    raise NotImplementedError("write your pallas kernel here")


# speedup vs baseline: 2.3352x; 2.3352x over previous
import functools

import jax
import jax.numpy as jnp
from jax.experimental import pallas as pl
from jax.experimental.pallas import tpu as pltpu

E, N, D = 8, 4096, 1024
TN = 2048


def _moe_body(x_ref, wr_ref, br_ref, we_ref, be_ref, out_ref, gn_ref):
    e = pl.program_id(1)
    xb = x_ref[0]

    @pl.when(e == 0)
    def _init():
        logits = (
            jnp.dot(xb, wr_ref[...], preferred_element_type=jnp.float32)
            + br_ref[...]
        )
        m = jnp.max(logits, axis=-1, keepdims=True)
        ex = jnp.exp(logits - m)
        gates = ex / jnp.sum(ex, axis=-1, keepdims=True)
        gn = gates / jnp.sum(gates, axis=-1, keepdims=True)
        gn_ref[...] = gn
        out_ref[...] = jnp.dot(gn, be_ref[...], preferred_element_type=jnp.float32)

    onehot = (jax.lax.broadcasted_iota(jnp.int32, (1, E), 1) == e).astype(
        jnp.float32
    )
    gcol = jnp.sum(gn_ref[...] * onehot, axis=-1, keepdims=True)

    partial = jnp.dot(
        xb.astype(jnp.bfloat16),
        we_ref[0].astype(jnp.bfloat16),
        preferred_element_type=jnp.float32,
    )
    out_ref[...] += gcol * partial


@jax.jit
def _moe(x, Wr, br, We, be):
    num_tiles = N // TN
    grid = (num_tiles, E)
    return pl.pallas_call(
        _moe_body,
        grid=grid,
        in_specs=[
            pl.BlockSpec((1, TN, D), lambda nt, e: (e, nt, 0)),
            pl.BlockSpec((D, E), lambda nt, e: (0, 0)),
            pl.BlockSpec((1, E), lambda nt, e: (0, 0)),
            pl.BlockSpec((1, D, D), lambda nt, e: (e, 0, 0)),
            pl.BlockSpec((E, D), lambda nt, e: (0, 0)),
        ],
        out_specs=pl.BlockSpec((TN, D), lambda nt, e: (nt, 0)),
        out_shape=jax.ShapeDtypeStruct((N, D), jnp.float32),
        scratch_shapes=[pltpu.VMEM((TN, E), jnp.float32)],
        compiler_params=pltpu.CompilerParams(
            dimension_semantics=("parallel", "arbitrary"),
        ),
    )(x, Wr, br, We, be)


def kernel(x, Wr, br, We, be):
    return _moe(x, Wr, br.reshape(1, E), We, be)


# speedup vs baseline: 2.3558x; 1.0088x over previous
import functools

import jax
import jax.numpy as jnp
from jax.experimental import pallas as pl
from jax.experimental.pallas import tpu as pltpu

E, N, D = 8, 4096, 1024
TN = 2048


def _moe_body(x_ref, wr_ref, br_ref, we_ref, be_ref, out_ref, gn_ref):
    e = pl.program_id(1)
    xb = x_ref[0]

    @pl.when(e == 0)
    def _init():
        logits = (
            jnp.dot(xb, wr_ref[...], preferred_element_type=jnp.float32)
            + br_ref[...]
        )
        m = jnp.max(logits, axis=-1, keepdims=True)
        ex = jnp.exp(logits - m)
        gates = ex / jnp.sum(ex, axis=-1, keepdims=True)
        gn = gates / jnp.sum(gates, axis=-1, keepdims=True)
        gn_ref[...] = gn
        out_ref[...] = jnp.dot(gn, be_ref[...], preferred_element_type=jnp.float32)

    onehot = (jax.lax.broadcasted_iota(jnp.int32, (1, E), 1) == e).astype(
        jnp.float32
    )
    gcol = jnp.sum(gn_ref[...] * onehot, axis=-1, keepdims=True)

    partial = jnp.dot(xb, we_ref[0], preferred_element_type=jnp.float32)
    out_ref[...] += gcol * partial


@jax.jit
def _moe(x, Wr, br, We, be):
    num_tiles = N // TN
    grid = (num_tiles, E)
    return pl.pallas_call(
        _moe_body,
        grid=grid,
        in_specs=[
            pl.BlockSpec((1, TN, D), lambda nt, e: (e, nt, 0)),
            pl.BlockSpec((D, E), lambda nt, e: (0, 0)),
            pl.BlockSpec((1, E), lambda nt, e: (0, 0)),
            pl.BlockSpec((1, D, D), lambda nt, e: (e, 0, 0)),
            pl.BlockSpec((E, D), lambda nt, e: (0, 0)),
        ],
        out_specs=pl.BlockSpec((TN, D), lambda nt, e: (nt, 0)),
        out_shape=jax.ShapeDtypeStruct((N, D), jnp.float32),
        scratch_shapes=[pltpu.VMEM((TN, E), jnp.float32)],
        compiler_params=pltpu.CompilerParams(
            dimension_semantics=("parallel", "arbitrary"),
        ),
    )(x, Wr, br, We, be)


def kernel(x, Wr, br, We, be):
    return _moe(x, Wr, br.reshape(1, E), We, be)


# speedup vs baseline: 2.4793x; 1.0524x over previous
import jax
import jax.numpy as jnp
from jax.experimental import pallas as pl
from jax.experimental.pallas import tpu as pltpu

E, N, D = 8, 4096, 1024
TN = 256


def _moe_body(x_ref, wr_ref, br_ref, we_ref, be_ref, out_ref):
    x0 = x_ref[0]
    logits = (
        jnp.dot(x0, wr_ref[...], preferred_element_type=jnp.float32)
        + br_ref[...]
    )
    m = jnp.max(logits, axis=-1, keepdims=True)
    ex = jnp.exp(logits - m)
    gates = ex / jnp.sum(ex, axis=-1, keepdims=True)
    gn = gates / jnp.sum(gates, axis=-1, keepdims=True)

    acc = jnp.dot(gn, be_ref[...], preferred_element_type=jnp.float32)
    for e in range(E):
        partial = jnp.dot(
            x_ref[e], we_ref[e], preferred_element_type=jnp.float32
        )
        acc = acc + gn[:, e : e + 1] * partial
    out_ref[...] = acc


@jax.jit
def _moe(x, Wr, br, We, be):
    num_tiles = N // TN
    return pl.pallas_call(
        _moe_body,
        grid=(num_tiles,),
        in_specs=[
            pl.BlockSpec((E, TN, D), lambda nt: (0, nt, 0)),
            pl.BlockSpec((D, E), lambda nt: (0, 0)),
            pl.BlockSpec((1, E), lambda nt: (0, 0)),
            pl.BlockSpec((E, D, D), lambda nt: (0, 0, 0)),
            pl.BlockSpec((E, D), lambda nt: (0, 0)),
        ],
        out_specs=pl.BlockSpec((TN, D), lambda nt: (nt, 0)),
        out_shape=jax.ShapeDtypeStruct((N, D), jnp.float32),
        compiler_params=pltpu.CompilerParams(
            dimension_semantics=("arbitrary",),
        ),
    )(x, Wr, br, We, be)


def kernel(x, Wr, br, We, be):
    return _moe(x, Wr, br.reshape(1, E), We, be)
